# baseline (speedup 1.0000x reference)
import functools
import jax
import jax.numpy as jnp
from jax import lax
from jax.experimental import pallas as pl
from jax.experimental.pallas import tpu as pltpu, tpu_sc as plsc

VOCAB_D = 1024
SEQ_LEN = 2048
N_BATCH = 16
P_LEN = 40
N_REL1 = 238

C = 32
N_CHUNK = 1024 // C
DEPTH = 3


def _body(tokens_hbm, wte_hbm, prompts_hbm, out_hbm,
          idxc0, idxc1, idxc2, buf0, buf1, buf2, tok16, idxpa, idxpb,
          gs0, gs1, gs2, ws0, ws1, ws2):
    p = lax.axis_index("c")
    b = lax.axis_index("s")
    base = b * SEQ_LEN + P_LEN + p * (1024 - P_LEN)

    idxcs = (idxc0, idxc1, idxc2)
    bufs = (buf0, buf1, buf2)
    gss = (gs0, gs1, gs2)
    wss = (ws0, ws1, ws2)

    def fire_gather(c, s):
        pltpu.sync_copy(tokens_hbm.at[pl.ds(base + c * C, C)], idxcs[s])
        pltpu.async_copy(wte_hbm.at[idxcs[s]], bufs[s], gss[s])

    def wait_gather(s):
        pltpu.make_async_copy(wte_hbm.at[idxcs[s]], bufs[s], gss[s]).wait()

    def fire_write(c, s):
        pltpu.async_copy(bufs[s], out_hbm.at[pl.ds(base + c * C, C)], wss[s])

    def wait_write(s):
        pltpu.make_async_copy(bufs[s], out_hbm.at[pl.ds(base, C)],
                              wss[s]).wait()

    @pl.when(p == 0)
    def _prompt_phase():
        pltpu.sync_copy(tokens_hbm.at[pl.ds(b * SEQ_LEN + P_LEN, 16)], tok16)
        tv = tok16[pl.ds(0, 16)]
        r = (tv[1] % N_REL1) * P_LEN
        io = lax.iota(jnp.int32, 16)
        idxpa[pl.ds(0, 16)] = io + r
        idxpa[pl.ds(16, 16)] = io + (r + 16)
        idxpb[pl.ds(0, 16)] = jnp.minimum(io + 32, P_LEN - 1) + r
        idxpb[pl.ds(16, 16)] = jnp.minimum(io + 48, P_LEN - 1) + r
        pltpu.async_copy(prompts_hbm.at[idxpa], buf0, gs0)
        pltpu.async_copy(prompts_hbm.at[idxpb], buf1, gs1)
        pltpu.make_async_copy(prompts_hbm.at[idxpa], buf0, gs0).wait()
        pltpu.make_async_copy(prompts_hbm.at[idxpb], buf1, gs1).wait()
        pltpu.sync_copy(buf0, out_hbm.at[pl.ds(b * SEQ_LEN, C)])
        pltpu.sync_copy(buf1.at[pl.ds(0, P_LEN - C)],
                        out_hbm.at[pl.ds(b * SEQ_LEN + C, P_LEN - C)])

    fire_gather(0, 0)
    fire_gather(1, 1)
    fire_gather(2, 2)

    def ring_iter(j, carry):
        for s in range(DEPTH):
            cw = DEPTH * j + s

            @pl.when(cw < N_CHUNK)
            def _():
                wait_gather(s)
                fire_write(cw, s)
        for s in range(DEPTH):
            cn = DEPTH * j + DEPTH + s

            @pl.when(cn < N_CHUNK)
            def _():
                wait_write(s)
                fire_gather(cn, s)
        return carry

    n_iter = (N_CHUNK + DEPTH - 1) // DEPTH
    lax.fori_loop(0, n_iter, ring_iter, 0)
    for s in range(DEPTH):
        wait_write(s)


@functools.partial(
    pl.kernel,
    out_type=jax.ShapeDtypeStruct((N_BATCH * SEQ_LEN, VOCAB_D), jnp.float32),
    mesh=plsc.VectorSubcoreMesh(core_axis_name="c", subcore_axis_name="s"),
    scratch_types=[
        pltpu.VMEM((C,), jnp.int32),
        pltpu.VMEM((C,), jnp.int32),
        pltpu.VMEM((C,), jnp.int32),
        pltpu.VMEM((C, VOCAB_D), jnp.float32),
        pltpu.VMEM((C, VOCAB_D), jnp.float32),
        pltpu.VMEM((C, VOCAB_D), jnp.float32),
        pltpu.VMEM((16,), jnp.int32),
        pltpu.VMEM((32,), jnp.int32),
        pltpu.VMEM((32,), jnp.int32),
        pltpu.SemaphoreType.DMA,
        pltpu.SemaphoreType.DMA,
        pltpu.SemaphoreType.DMA,
        pltpu.SemaphoreType.DMA,
        pltpu.SemaphoreType.DMA,
        pltpu.SemaphoreType.DMA,
    ],
)
def _gather_kernel(tokens_hbm, wte_hbm, prompts_hbm, out_hbm, *scratch):
    _body(tokens_hbm, wte_hbm, prompts_hbm, out_hbm, *scratch)


@jax.jit
def kernel(tokens, wte_weight, prompts):
    tokens_flat = tokens.reshape(-1)
    prompts_flat = prompts.reshape(N_REL1 * P_LEN, VOCAB_D)
    out = _gather_kernel(tokens_flat, wte_weight, prompts_flat)
    return out.reshape(N_BATCH, SEQ_LEN, VOCAB_D)


# speedup vs baseline: 1.0373x; 1.0373x over previous
import functools
import jax
import jax.numpy as jnp
from jax import lax
from jax.experimental import pallas as pl
from jax.experimental.pallas import tpu as pltpu, tpu_sc as plsc

VOCAB_D = 1024
SEQ_LEN = 2048
N_BATCH = 16
P_LEN = 40
N_REL1 = 238

C = 32
N_CHUNK = 1024 // C


def _body(tokens_hbm, wte_hbm, prompts_hbm, out_hbm,
          idx_v, buf0, buf1, tok16, idxp, pbuf,
          gs0, gs1, psem):
    p = lax.axis_index("c")
    b = lax.axis_index("s")
    base = b * SEQ_LEN + P_LEN + p * (1024 - P_LEN)

    pltpu.sync_copy(tokens_hbm.at[pl.ds(base, 1024)], idx_v)

    def start_chunk(i, buf, sem):
        pltpu.async_copy(wte_hbm.at[idx_v.at[pl.ds(i * C, C)]], buf, sem)

    def wait_chunk(i, buf, sem):
        pltpu.make_async_copy(wte_hbm.at[idx_v.at[pl.ds(i * C, C)]],
                              buf, sem).wait()

    start_chunk(0, buf0, gs0)

    @pl.when(p == 0)
    def _prompt_phase():
        pltpu.sync_copy(tokens_hbm.at[pl.ds(b * SEQ_LEN + P_LEN, 16)], tok16)
        tv = tok16[pl.ds(0, 16)]
        r = (tv[1] % N_REL1) * P_LEN
        for k in range(3):
            v = jnp.minimum(lax.iota(jnp.int32, 16) + (16 * k), P_LEN - 1)
            idxp[pl.ds(16 * k, 16)] = v + r
        pltpu.async_copy(prompts_hbm.at[idxp], pbuf, psem).wait()
        pltpu.sync_copy(pbuf.at[pl.ds(0, P_LEN)],
                        out_hbm.at[pl.ds(b * SEQ_LEN, P_LEN)])

    def loop_body(j, carry):
        start_chunk(2 * j + 1, buf1, gs1)
        wait_chunk(2 * j, buf0, gs0)
        pltpu.sync_copy(buf0, out_hbm.at[pl.ds(base + (2 * j) * C, C)])

        @pl.when(j < N_CHUNK // 2 - 1)
        def _():
            start_chunk(2 * j + 2, buf0, gs0)

        wait_chunk(2 * j + 1, buf1, gs1)
        pltpu.sync_copy(buf1, out_hbm.at[pl.ds(base + (2 * j + 1) * C, C)])
        return carry

    lax.fori_loop(0, N_CHUNK // 2, loop_body, 0)


@functools.partial(
    pl.kernel,
    out_type=jax.ShapeDtypeStruct((N_BATCH * SEQ_LEN, VOCAB_D), jnp.float32),
    mesh=plsc.VectorSubcoreMesh(core_axis_name="c", subcore_axis_name="s"),
    scratch_types=[
        pltpu.VMEM((1024,), jnp.int32),
        pltpu.VMEM((C, VOCAB_D), jnp.float32),
        pltpu.VMEM((C, VOCAB_D), jnp.float32),
        pltpu.VMEM((16,), jnp.int32),
        pltpu.VMEM((48,), jnp.int32),
        pltpu.VMEM((48, VOCAB_D), jnp.float32),
        pltpu.SemaphoreType.DMA,
        pltpu.SemaphoreType.DMA,
        pltpu.SemaphoreType.DMA,
    ],
)
def _gather_kernel(tokens_hbm, wte_hbm, prompts_hbm, out_hbm, *scratch):
    _body(tokens_hbm, wte_hbm, prompts_hbm, out_hbm, *scratch)


@jax.jit
def kernel(tokens, wte_weight, prompts):
    tokens_flat = tokens.reshape(-1)
    prompts_flat = prompts.reshape(N_REL1 * P_LEN, VOCAB_D)
    out = _gather_kernel(tokens_flat, wte_weight, prompts_flat)
    return out.reshape(N_BATCH, SEQ_LEN, VOCAB_D)
